# baseline (speedup 1.0000x reference)
import dataclasses
import functools

import jax
import jax.numpy as jnp
from jax import lax
from jax.experimental import pallas as pl
from jax.experimental.pallas import tpu as pltpu
from jax.experimental.pallas import tpu_sc as plsc

C = 32
H = 512
W = 512

NC = 2
NS = 16
NW = NC * NS
L = 16

CHUNK = 512
SUB = 128
NSUB = CHUNK // SUB
GROUPS = CHUNK // L
UNROLL = 8


def _compiler_params():
    cp = pltpu.CompilerParams(use_tc_tiling_on_sc=False)
    if "needs_layout_passes" in pltpu.CompilerParams.__dataclass_fields__:
        cp = dataclasses.replace(cp, needs_layout_passes=False)
    return cp


HB = 8


def _transpose_body(p_ref, t_ref):
    blk = p_ref[0].reshape(C, HB * W)
    t_ref[...] = jnp.transpose(blk, (1, 0))


def _to_table(plane):
    return pl.pallas_call(
        _transpose_body,
        grid=(H // HB,),
        in_specs=[pl.BlockSpec((1, C, HB, W), lambda i: (0, 0, i, 0))],
        out_specs=pl.BlockSpec((HB * W, C), lambda i: (i, 0)),
        out_shape=jax.ShapeDtypeStruct((H * W, C), jnp.float32),
    )(plane)


@functools.cache
def _make_sc_lookup(n):
    npw = n // NW
    chunks = npw // CHUNK
    mesh = plsc.VectorSubcoreMesh(core_axis_name="c", subcore_axis_name="s")

    @functools.partial(
        pl.kernel,
        out_type=jax.ShapeDtypeStruct((n, C), jnp.float32),
        mesh=mesh,
        compiler_params=_compiler_params(),
        scratch_types=[
            pltpu.VMEM((2 * CHUNK,), jnp.float32),
            pltpu.VMEM((CHUNK,), jnp.int32),
            pltpu.VMEM((CHUNK,), jnp.int32),
            pltpu.VMEM((CHUNK,), jnp.int32),
            pltpu.VMEM((CHUNK,), jnp.int32),
            pltpu.VMEM((CHUNK,), jnp.float32),
            pltpu.VMEM((CHUNK,), jnp.float32),
            pltpu.VMEM((CHUNK,), jnp.float32),
            pltpu.VMEM((CHUNK,), jnp.float32),
            pltpu.VMEM((CHUNK, C), jnp.float32),
            pltpu.VMEM((CHUNK, C), jnp.float32),
            pltpu.VMEM((CHUNK, C), jnp.float32),
            pltpu.VMEM((CHUNK, C), jnp.float32),
            pltpu.VMEM((CHUNK, C), jnp.float32),
            pltpu.SemaphoreType.DMA,
        ],
    )
    def lookup(xy_hbm, table_hbm, out_hbm, xy_v,
               i00_v, i01_v, i10_v, i11_v, w00_v, w01_v, w10_v, w11_v,
               t00_v, t01_v, t10_v, t11_v, o_v, sem):
        wid = lax.axis_index("s") * NC + lax.axis_index("c")
        base = wid * npw
        iota2 = lax.iota(jnp.int32, L) * 2

        @pl.loop(0, chunks)
        def _chunk(k):
            off = base + k * CHUNK
            pltpu.sync_copy(xy_hbm.at[pl.ds(2 * off, 2 * CHUNK)], xy_v)

            @pl.loop(0, GROUPS)
            def _pass1(g):
                s = pl.ds(g * L, L)
                exi = g * (2 * L) + iota2
                ix = (plsc.load_gather(xy_v, [exi]) + 1.0) * 0.5 * (W - 1)
                iy = (plsc.load_gather(xy_v, [exi + 1]) + 1.0) * 0.5 * (H - 1)
                x0 = jnp.minimum(ix.astype(jnp.int32), W - 2)
                y0 = jnp.minimum(iy.astype(jnp.int32), H - 2)
                fx = ix - x0.astype(jnp.float32)
                fy = iy - y0.astype(jnp.float32)
                i00 = y0 * W + x0
                i00_v[s] = i00
                i01_v[s] = i00 + 1
                i10_v[s] = i00 + W
                i11_v[s] = i00 + (W + 1)
                gx = 1.0 - fx
                gy = 1.0 - fy
                w00_v[s] = gy * gx
                w01_v[s] = gy * fx
                w10_v[s] = fy * gx
                w11_v[s] = fy * fx

            copies = []
            for t_v, i_v in ((t00_v, i00_v), (t01_v, i01_v),
                             (t10_v, i10_v), (t11_v, i11_v)):
                for u in range(NSUB):
                    sl = pl.ds(u * SUB, SUB)
                    copies.append(pltpu.async_copy(
                        table_hbm.at[i_v.at[sl]], t_v.at[sl], sem))
            for cp in copies:
                cp.wait()

            @pl.loop(0, CHUNK, step=UNROLL)
            def _pass2(p0):
                for dp in range(UNROLL):
                    p = p0 + dp
                    pv = jnp.full((L,), 0, jnp.int32) + p
                    w00 = plsc.load_gather(w00_v, [pv])
                    w01 = plsc.load_gather(w01_v, [pv])
                    w10 = plsc.load_gather(w10_v, [pv])
                    w11 = plsc.load_gather(w11_v, [pv])
                    for h in range(C // L):
                        s = pl.ds(h * L, L)
                        v = (w00 * t00_v[p, s] + w01 * t01_v[p, s]
                             + w10 * t10_v[p, s] + w11 * t11_v[p, s])
                        o_v[p, s] = v

            pltpu.sync_copy(o_v, out_hbm.at[pl.ds(off, CHUNK)])

    return lookup


def kernel(x, plane):
    lead = x.shape[:-1]
    xy = x.reshape(-1)
    n = xy.shape[0] // 2
    table = _to_table(plane)
    out = _make_sc_lookup(n)(xy, table)
    return out.reshape(lead + (C,))


# speedup vs baseline: 1.7791x; 1.7791x over previous
import dataclasses
import functools

import jax
import jax.numpy as jnp
from jax import lax
from jax.experimental import pallas as pl
from jax.experimental.pallas import tpu as pltpu
from jax.experimental.pallas import tpu_sc as plsc

C = 32
H = 512
W = 512

NC = 2
NS = 16
NW = NC * NS
L = 16

CHUNK = 512
SUB = 128
NSUB = CHUNK // SUB
GROUPS = CHUNK // L
UNROLL = 8


def _compiler_params():
    cp = pltpu.CompilerParams(use_tc_tiling_on_sc=False)
    if "needs_layout_passes" in pltpu.CompilerParams.__dataclass_fields__:
        cp = dataclasses.replace(cp, needs_layout_passes=False)
    return cp


@functools.cache
def _make_sc_lookup(n):
    npw = n // NW
    chunks = npw // CHUNK
    mesh = plsc.VectorSubcoreMesh(core_axis_name="c", subcore_axis_name="s")

    @functools.partial(
        pl.kernel,
        out_type=jax.ShapeDtypeStruct((n * C,), jnp.float32),
        mesh=mesh,
        compiler_params=_compiler_params(),
        scratch_types=[
            pltpu.VMEM((CHUNK,), jnp.float32),
            pltpu.VMEM((CHUNK,), jnp.float32),
            pltpu.VMEM((CHUNK,), jnp.int32),
            pltpu.VMEM((CHUNK,), jnp.int32),
            pltpu.VMEM((CHUNK,), jnp.int32),
            pltpu.VMEM((CHUNK,), jnp.int32),
            pltpu.VMEM((CHUNK,), jnp.float32),
            pltpu.VMEM((CHUNK,), jnp.float32),
            pltpu.VMEM((CHUNK,), jnp.float32),
            pltpu.VMEM((CHUNK,), jnp.float32),
            pltpu.VMEM((CHUNK, C), jnp.float32),
            pltpu.VMEM((CHUNK, C), jnp.float32),
            pltpu.VMEM((CHUNK, C), jnp.float32),
            pltpu.VMEM((CHUNK, C), jnp.float32),
            pltpu.VMEM((CHUNK * C,), jnp.float32),
            pltpu.SemaphoreType.DMA,
        ],
    )
    def lookup(xs_hbm, ys_hbm, table_hbm, out_hbm, xs_v, ys_v,
               i00_v, i01_v, i10_v, i11_v, w00_v, w01_v, w10_v, w11_v,
               t00_v, t01_v, t10_v, t11_v, o_v, sem):
        wid = lax.axis_index("s") * NC + lax.axis_index("c")
        base = wid * npw

        @pl.loop(0, chunks)
        def _chunk(k):
            off = base + k * CHUNK
            pltpu.sync_copy(xs_hbm.at[pl.ds(off, CHUNK)], xs_v)
            pltpu.sync_copy(ys_hbm.at[pl.ds(off, CHUNK)], ys_v)

            @pl.loop(0, GROUPS)
            def _pass1(g):
                s = pl.ds(g * L, L)
                ix = (xs_v[s] + 1.0) * 0.5 * (W - 1)
                iy = (ys_v[s] + 1.0) * 0.5 * (H - 1)
                x0 = jnp.minimum(ix.astype(jnp.int32), W - 2)
                y0 = jnp.minimum(iy.astype(jnp.int32), H - 2)
                fx = ix - x0.astype(jnp.float32)
                fy = iy - y0.astype(jnp.float32)
                i00 = y0 * W + x0
                i00_v[s] = i00
                i01_v[s] = i00 + 1
                i10_v[s] = i00 + W
                i11_v[s] = i00 + (W + 1)
                gx = 1.0 - fx
                gy = 1.0 - fy
                w00_v[s] = gy * gx
                w01_v[s] = gy * fx
                w10_v[s] = fy * gx
                w11_v[s] = fy * fx

            copies = []
            for t_v, i_v in ((t00_v, i00_v), (t01_v, i01_v),
                             (t10_v, i10_v), (t11_v, i11_v)):
                for u in range(NSUB):
                    sl = pl.ds(u * SUB, SUB)
                    copies.append(pltpu.async_copy(
                        table_hbm.at[i_v.at[sl]], t_v.at[sl], sem))
            for cp in copies:
                cp.wait()

            @pl.loop(0, CHUNK, step=UNROLL)
            def _pass2(p0):
                for dp in range(UNROLL):
                    p = p0 + dp
                    pv = jnp.full((L,), 0, jnp.int32) + p
                    w00 = plsc.load_gather(w00_v, [pv])
                    w01 = plsc.load_gather(w01_v, [pv])
                    w10 = plsc.load_gather(w10_v, [pv])
                    w11 = plsc.load_gather(w11_v, [pv])
                    for h in range(C // L):
                        s = pl.ds(h * L, L)
                        v = (w00 * t00_v[p, s] + w01 * t01_v[p, s]
                             + w10 * t10_v[p, s] + w11 * t11_v[p, s])
                        o_v[pl.ds(p * C + h * L, L)] = v

            pltpu.sync_copy(o_v, out_hbm.at[pl.ds(off * C, CHUNK * C)])

    return lookup


def kernel(x, plane):
    lead = x.shape[:-1]
    coords = x.reshape(-1, 2)
    n = coords.shape[0]
    xs = coords[:, 0]
    ys = coords[:, 1]
    table = jnp.transpose(plane[0], (1, 2, 0)).reshape(H * W, C)
    out = _make_sc_lookup(n)(xs, ys, table)
    return out.reshape(lead + (C,))


# speedup vs baseline: 2.0327x; 1.1425x over previous
import dataclasses
import functools

import jax
import jax.numpy as jnp
from jax import lax
from jax.experimental import pallas as pl
from jax.experimental.pallas import tpu as pltpu
from jax.experimental.pallas import tpu_sc as plsc

C = 32
H = 512
W = 512

NC = 2
NS = 16
NW = NC * NS
L = 16

CHUNK = 256
SUB = 128
NSUB = CHUNK // SUB
GROUPS = CHUNK // L
UNROLL = 8
NBUF = 2


def _compiler_params():
    cp = pltpu.CompilerParams(use_tc_tiling_on_sc=False)
    if "needs_layout_passes" in pltpu.CompilerParams.__dataclass_fields__:
        cp = dataclasses.replace(cp, needs_layout_passes=False)
    return cp


def _set_scratch():
    return [
        pltpu.VMEM((CHUNK,), jnp.float32),
        pltpu.VMEM((CHUNK,), jnp.float32),
        pltpu.VMEM((CHUNK,), jnp.int32),
        pltpu.VMEM((CHUNK,), jnp.int32),
        pltpu.VMEM((CHUNK,), jnp.int32),
        pltpu.VMEM((CHUNK,), jnp.int32),
        pltpu.VMEM((CHUNK,), jnp.float32),
        pltpu.VMEM((CHUNK,), jnp.float32),
        pltpu.VMEM((CHUNK,), jnp.float32),
        pltpu.VMEM((CHUNK,), jnp.float32),
        pltpu.VMEM((CHUNK, C), jnp.float32),
        pltpu.VMEM((CHUNK, C), jnp.float32),
        pltpu.VMEM((CHUNK, C), jnp.float32),
        pltpu.VMEM((CHUNK, C), jnp.float32),
        pltpu.VMEM((CHUNK * C,), jnp.float32),
    ]


NSET = len(_set_scratch())


@functools.cache
def _make_sc_lookup(n):
    npw = n // NW
    chunks = npw // CHUNK
    assert chunks >= 4 and chunks % 2 == 0
    mesh = plsc.VectorSubcoreMesh(core_axis_name="c", subcore_axis_name="s")

    @functools.partial(
        pl.kernel,
        out_type=jax.ShapeDtypeStruct((n * C,), jnp.float32),
        mesh=mesh,
        compiler_params=_compiler_params(),
        scratch_types=_set_scratch() + _set_scratch() + [
            pltpu.SemaphoreType.DMA,
            pltpu.SemaphoreType.DMA,
            pltpu.SemaphoreType.DMA,
            pltpu.SemaphoreType.DMA,
        ],
    )
    def lookup(xs_hbm, ys_hbm, table_hbm, out_hbm, *scr):
        sets = [scr[:NSET], scr[NSET:2 * NSET]]
        sem_g = [scr[2 * NSET], scr[2 * NSET + 1]]
        sem_o = [scr[2 * NSET + 2], scr[2 * NSET + 3]]
        wid = lax.axis_index("s") * NC + lax.axis_index("c")
        base = wid * npw

        def bufs(st):
            (xs_v, ys_v, i00_v, i01_v, i10_v, i11_v,
             w00_v, w01_v, w10_v, w11_v,
             t00_v, t01_v, t10_v, t11_v, o_v) = sets[st]
            return (xs_v, ys_v, (i00_v, i01_v, i10_v, i11_v),
                    (w00_v, w01_v, w10_v, w11_v),
                    (t00_v, t01_v, t10_v, t11_v), o_v)

        def stage(k, st):
            xs_v, ys_v, i_vs, w_vs, t_vs, _ = bufs(st)
            off = base + k * CHUNK
            pltpu.sync_copy(xs_hbm.at[pl.ds(off, CHUNK)], xs_v)
            pltpu.sync_copy(ys_hbm.at[pl.ds(off, CHUNK)], ys_v)

            @pl.loop(0, GROUPS)
            def _pass1(g):
                s = pl.ds(g * L, L)
                ix = (xs_v[s] + 1.0) * 0.5 * (W - 1)
                iy = (ys_v[s] + 1.0) * 0.5 * (H - 1)
                x0 = jnp.minimum(ix.astype(jnp.int32), W - 2)
                y0 = jnp.minimum(iy.astype(jnp.int32), H - 2)
                fx = ix - x0.astype(jnp.float32)
                fy = iy - y0.astype(jnp.float32)
                i00 = y0 * W + x0
                i_vs[0][s] = i00
                i_vs[1][s] = i00 + 1
                i_vs[2][s] = i00 + W
                i_vs[3][s] = i00 + (W + 1)
                gx = 1.0 - fx
                gy = 1.0 - fy
                w_vs[0][s] = gy * gx
                w_vs[1][s] = gy * fx
                w_vs[2][s] = fy * gx
                w_vs[3][s] = fy * fx

            for t_v, i_v in zip(t_vs, i_vs):
                for u in range(NSUB):
                    sl = pl.ds(u * SUB, SUB)
                    pltpu.async_copy(table_hbm.at[i_v.at[sl]],
                                     t_v.at[sl], sem_g[st])

        def wait_gathers(st):
            _, _, i_vs, _, t_vs, _ = bufs(st)
            for t_v, i_v in zip(t_vs, i_vs):
                for u in range(NSUB):
                    sl = pl.ds(u * SUB, SUB)
                    pltpu.make_async_copy(table_hbm.at[i_v.at[sl]],
                                          t_v.at[sl], sem_g[st]).wait()

        def wait_out(k_prev, st):
            o_v = bufs(st)[5]
            off = base + k_prev * CHUNK
            pltpu.make_async_copy(
                o_v, out_hbm.at[pl.ds(off * C, CHUNK * C)], sem_o[st]).wait()

        def pass2_and_emit(k, st):
            _, _, _, w_vs, t_vs, o_v = bufs(st)
            t00_v, t01_v, t10_v, t11_v = t_vs

            @pl.loop(0, CHUNK, step=UNROLL)
            def _pass2(p0):
                for dp in range(UNROLL):
                    p = p0 + dp
                    pv = jnp.full((L,), 0, jnp.int32) + p
                    w00 = plsc.load_gather(w_vs[0], [pv])
                    w01 = plsc.load_gather(w_vs[1], [pv])
                    w10 = plsc.load_gather(w_vs[2], [pv])
                    w11 = plsc.load_gather(w_vs[3], [pv])
                    for h in range(C // L):
                        s = pl.ds(h * L, L)
                        v = (w00 * t00_v[p, s] + w01 * t01_v[p, s]
                             + w10 * t10_v[p, s] + w11 * t11_v[p, s])
                        o_v[pl.ds(p * C + h * L, L)] = v

            off = base + k * CHUNK
            pltpu.async_copy(o_v, out_hbm.at[pl.ds(off * C, CHUNK * C)],
                             sem_o[st])

        stage(0, 0)
        stage(1, 1)

        @pl.loop(0, chunks - 2, step=2)
        def _main(j):
            for st in range(NBUF):
                k = j + st
                wait_gathers(st)

                @pl.when(k >= 2)
                def _():
                    wait_out(k - 2, st)

                pass2_and_emit(k, st)
                stage(k + 2, st)

        for st in range(NBUF):
            k = chunks - 2 + st
            wait_gathers(st)
            wait_out(k - 2, st)
            pass2_and_emit(k, st)
        for st in range(NBUF):
            wait_out(chunks - 2 + st, st)

    return lookup


def kernel(x, plane):
    lead = x.shape[:-1]
    coords = x.reshape(-1, 2)
    n = coords.shape[0]
    xs = coords[:, 0]
    ys = coords[:, 1]
    table = jnp.transpose(plane[0], (1, 2, 0)).reshape(H * W, C)
    out = _make_sc_lookup(n)(xs, ys, table)
    return out.reshape(lead + (C,))
